# baseline (speedup 1.0000x reference)
import numpy as np
import jax
import jax.numpy as jnp
from jax import lax
from jax.experimental import pallas as pl
from jax.experimental.pallas import tpu as pltpu

_B, _C, _L = 256, 20, 4096
_TL = 2048

_ROTS = ((13, 15, 26, 6), (17, 29, 16, 24))


def _np_threefry2x32(k0, k1, x0, x1):
    mask = np.uint64(0xFFFFFFFF)
    x0 = np.asarray(x0, np.uint64)
    x1 = np.asarray(x1, np.uint64)

    def rotl(x, d):
        return ((x << np.uint64(d)) | (x >> np.uint64(32 - d))) & mask

    ks0, ks1 = np.uint64(k0), np.uint64(k1)
    ks2 = ks0 ^ ks1 ^ np.uint64(0x1BD11BDA)
    x0 = (x0 + ks0) & mask
    x1 = (x1 + ks1) & mask
    inj = ((ks1, ks2, 1), (ks2, ks0, 2), (ks0, ks1, 3), (ks1, ks2, 4), (ks2, ks0, 5))
    for i, (a, b, inc) in enumerate(inj):
        for r in _ROTS[i % 2]:
            x0 = (x0 + x1) & mask
            x1 = rotl(x1, r)
            x1 = x1 ^ x0
        x0 = (x0 + a) & mask
        x1 = (x1 + b + np.uint64(inc)) & mask
    return x0.astype(np.uint32), x1.astype(np.uint32)


_o0, _o1 = _np_threefry2x32(0, 42, [0, 0], [0, 1])
_KS0, _KS1 = int(_o0[0]), int(_o1[0])


def _sample_kernel(x_ref, o_ref):
    b = pl.program_id(0)
    lt = pl.program_id(1)

    l_iota = lax.broadcasted_iota(jnp.uint32, (_C, _TL), 1)
    c_iota = lax.broadcasted_iota(jnp.uint32, (_C, _TL), 0)
    base = (b * (_L * _C) + lt * (_TL * _C)).astype(jnp.uint32)
    x1 = base + l_iota * np.uint32(_C) + c_iota
    x0 = jnp.zeros_like(x1)

    ks0 = np.uint32(_KS0)
    ks1 = np.uint32(_KS1)
    ks2 = np.uint32(_KS0 ^ _KS1 ^ 0x1BD11BDA)

    def rotl(x, d):
        return lax.shift_left(x, np.uint32(d)) | lax.shift_right_logical(
            x, np.uint32(32 - d))

    x0 = x0 + ks0
    x1 = x1 + ks1
    inj = ((ks1, ks2, 1), (ks2, ks0, 2), (ks0, ks1, 3), (ks1, ks2, 4), (ks2, ks0, 5))
    for i, (a, b_, inc) in enumerate(inj):
        for r in _ROTS[i % 2]:
            x0 = x0 + x1
            x1 = rotl(x1, r)
            x1 = x1 ^ x0
        x0 = x0 + a
        x1 = x1 + b_ + np.uint32(inc)
    bits = x0 ^ x1

    fb = lax.shift_right_logical(bits, np.uint32(9)) | np.uint32(0x3F800000)
    u = lax.bitcast_convert_type(fb, jnp.float32) - np.float32(1.0)
    tiny = np.float32(np.finfo(np.float32).tiny)
    u = jnp.maximum(tiny, u * (np.float32(1.0) - tiny) + tiny)
    g = -jnp.log(-jnp.log(u))

    vals = x_ref[0] + g
    maxv = jnp.max(vals, axis=0, keepdims=True)
    cm = lax.broadcasted_iota(jnp.int32, (_C, _TL), 0)
    idx = jnp.min(jnp.where(vals == maxv, cm, _C), axis=0, keepdims=True)
    o_ref[0] = (cm == idx).astype(jnp.float32)


def kernel(logits):
    return pl.pallas_call(
        _sample_kernel,
        grid=(_B, _L // _TL),
        in_specs=[pl.BlockSpec((1, _C, _TL), lambda b, lt: (b, 0, lt))],
        out_specs=pl.BlockSpec((1, _C, _TL), lambda b, lt: (b, 0, lt)),
        out_shape=jax.ShapeDtypeStruct((_B, _C, _L), jnp.float32),
        compiler_params=pltpu.CompilerParams(
            dimension_semantics=("parallel", "parallel")),
    )(logits)


# speedup vs baseline: 1.0580x; 1.0580x over previous
import numpy as np
import jax
import jax.numpy as jnp
from jax import lax
from jax.experimental import pallas as pl
from jax.experimental.pallas import tpu as pltpu

_B, _C, _L = 256, 20, 4096
_TL = 4096

_ROTS = ((13, 15, 26, 6), (17, 29, 16, 24))


def _np_threefry2x32(k0, k1, x0, x1):
    mask = np.uint64(0xFFFFFFFF)
    x0 = np.asarray(x0, np.uint64)
    x1 = np.asarray(x1, np.uint64)

    def rotl(x, d):
        return ((x << np.uint64(d)) | (x >> np.uint64(32 - d))) & mask

    ks0, ks1 = np.uint64(k0), np.uint64(k1)
    ks2 = ks0 ^ ks1 ^ np.uint64(0x1BD11BDA)
    x0 = (x0 + ks0) & mask
    x1 = (x1 + ks1) & mask
    inj = ((ks1, ks2, 1), (ks2, ks0, 2), (ks0, ks1, 3), (ks1, ks2, 4), (ks2, ks0, 5))
    for i, (a, b, inc) in enumerate(inj):
        for r in _ROTS[i % 2]:
            x0 = (x0 + x1) & mask
            x1 = rotl(x1, r)
            x1 = x1 ^ x0
        x0 = (x0 + a) & mask
        x1 = (x1 + b + np.uint64(inc)) & mask
    return x0.astype(np.uint32), x1.astype(np.uint32)


_o0, _o1 = _np_threefry2x32(0, 42, [0, 0], [0, 1])
_KS0, _KS1 = int(_o0[0]), int(_o1[0])


def _sample_kernel(x_ref, o_ref):
    b = pl.program_id(0)
    lt = pl.program_id(1)

    l_iota = lax.broadcasted_iota(jnp.uint32, (_C, _TL), 1)
    c_iota = lax.broadcasted_iota(jnp.uint32, (_C, _TL), 0)
    base = (b * (_L * _C) + lt * (_TL * _C)).astype(jnp.uint32)
    x1 = base + l_iota * np.uint32(_C) + c_iota
    x0 = jnp.zeros_like(x1)

    ks0 = np.uint32(_KS0)
    ks1 = np.uint32(_KS1)
    ks2 = np.uint32(_KS0 ^ _KS1 ^ 0x1BD11BDA)

    def rotl(x, d):
        return lax.shift_left(x, np.uint32(d)) | lax.shift_right_logical(
            x, np.uint32(32 - d))

    x0 = x0 + ks0
    x1 = x1 + ks1
    inj = ((ks1, ks2, 1), (ks2, ks0, 2), (ks0, ks1, 3), (ks1, ks2, 4), (ks2, ks0, 5))
    for i, (a, b_, inc) in enumerate(inj):
        for r in _ROTS[i % 2]:
            x0 = x0 + x1
            x1 = rotl(x1, r)
            x1 = x1 ^ x0
        x0 = x0 + a
        x1 = x1 + b_ + np.uint32(inc)
    bits = x0 ^ x1

    fb = lax.shift_right_logical(bits, np.uint32(9)) | np.uint32(0x3F800000)
    u = lax.bitcast_convert_type(fb, jnp.float32) - np.float32(1.0)
    tiny = np.float32(np.finfo(np.float32).tiny)
    u = jnp.maximum(tiny, u * (np.float32(1.0) - tiny) + tiny)
    g = -jnp.log(-jnp.log(u))

    vals = x_ref[0] + g
    maxv = jnp.max(vals, axis=0, keepdims=True)
    cm = lax.broadcasted_iota(jnp.int32, (_C, _TL), 0)
    idx = jnp.min(jnp.where(vals == maxv, cm, _C), axis=0, keepdims=True)
    o_ref[0] = (cm == idx).astype(jnp.float32)


def kernel(logits):
    return pl.pallas_call(
        _sample_kernel,
        grid=(_B, _L // _TL),
        in_specs=[pl.BlockSpec((1, _C, _TL), lambda b, lt: (b, 0, lt))],
        out_specs=pl.BlockSpec((1, _C, _TL), lambda b, lt: (b, 0, lt)),
        out_shape=jax.ShapeDtypeStruct((_B, _C, _L), jnp.float32),
        compiler_params=pltpu.CompilerParams(
            dimension_semantics=("parallel", "parallel")),
    )(logits)


# speedup vs baseline: 1.0782x; 1.0191x over previous
import numpy as np
import jax
import jax.numpy as jnp
from jax import lax
from jax.experimental import pallas as pl
from jax.experimental.pallas import tpu as pltpu

_B, _C, _L = 256, 20, 4096
_TL = 4096
_BB = 4

_ROTS = ((13, 15, 26, 6), (17, 29, 16, 24))


def _np_threefry2x32(k0, k1, x0, x1):
    mask = np.uint64(0xFFFFFFFF)
    x0 = np.asarray(x0, np.uint64)
    x1 = np.asarray(x1, np.uint64)

    def rotl(x, d):
        return ((x << np.uint64(d)) | (x >> np.uint64(32 - d))) & mask

    ks0, ks1 = np.uint64(k0), np.uint64(k1)
    ks2 = ks0 ^ ks1 ^ np.uint64(0x1BD11BDA)
    x0 = (x0 + ks0) & mask
    x1 = (x1 + ks1) & mask
    inj = ((ks1, ks2, 1), (ks2, ks0, 2), (ks0, ks1, 3), (ks1, ks2, 4), (ks2, ks0, 5))
    for i, (a, b, inc) in enumerate(inj):
        for r in _ROTS[i % 2]:
            x0 = (x0 + x1) & mask
            x1 = rotl(x1, r)
            x1 = x1 ^ x0
        x0 = (x0 + a) & mask
        x1 = (x1 + b + np.uint64(inc)) & mask
    return x0.astype(np.uint32), x1.astype(np.uint32)


_o0, _o1 = _np_threefry2x32(0, 42, [0, 0], [0, 1])
_KS0, _KS1 = int(_o0[0]), int(_o1[0])


def _sample_kernel(x_ref, o_ref):
    b = pl.program_id(0)

    shp = (_BB, _C, _TL)
    b_iota = lax.broadcasted_iota(jnp.uint32, shp, 0)
    l_iota = lax.broadcasted_iota(jnp.uint32, shp, 2)
    c_iota = lax.broadcasted_iota(jnp.uint32, shp, 1)
    base = (b * (_BB * _L * _C)).astype(jnp.uint32)
    x1 = (base + b_iota * np.uint32(_L * _C)
          + l_iota * np.uint32(_C) + c_iota)
    x0 = jnp.zeros_like(x1)

    ks0 = np.uint32(_KS0)
    ks1 = np.uint32(_KS1)
    ks2 = np.uint32(_KS0 ^ _KS1 ^ 0x1BD11BDA)

    def rotl(x, d):
        return lax.shift_left(x, np.uint32(d)) | lax.shift_right_logical(
            x, np.uint32(32 - d))

    x0 = x0 + ks0
    x1 = x1 + ks1
    inj = ((ks1, ks2, 1), (ks2, ks0, 2), (ks0, ks1, 3), (ks1, ks2, 4), (ks2, ks0, 5))
    for i, (a, b_, inc) in enumerate(inj):
        for r in _ROTS[i % 2]:
            x0 = x0 + x1
            x1 = rotl(x1, r)
            x1 = x1 ^ x0
        x0 = x0 + a
        x1 = x1 + b_ + np.uint32(inc)
    bits = x0 ^ x1

    fb = lax.shift_right_logical(bits, np.uint32(9)) | np.uint32(0x3F800000)
    u = lax.bitcast_convert_type(fb, jnp.float32) - np.float32(1.0)
    tiny = np.float32(np.finfo(np.float32).tiny)
    u = jnp.maximum(tiny, u * (np.float32(1.0) - tiny) + tiny)
    g = -jnp.log(-jnp.log(u))

    vals = x_ref[...] + g
    maxv = jnp.max(vals, axis=1, keepdims=True)
    cm = lax.broadcasted_iota(jnp.int32, shp, 1)
    idx = jnp.min(jnp.where(vals == maxv, cm, _C), axis=1, keepdims=True)
    o_ref[...] = (cm == idx).astype(jnp.float32)


def kernel(logits):
    return pl.pallas_call(
        _sample_kernel,
        grid=(_B // _BB,),
        in_specs=[pl.BlockSpec((_BB, _C, _TL), lambda b: (b, 0, 0))],
        out_specs=pl.BlockSpec((_BB, _C, _TL), lambda b: (b, 0, 0)),
        out_shape=jax.ShapeDtypeStruct((_B, _C, _L), jnp.float32),
        compiler_params=pltpu.CompilerParams(
            dimension_semantics=("parallel",)),
    )(logits)


# speedup vs baseline: 1.0909x; 1.0118x over previous
import numpy as np
import jax
import jax.numpy as jnp
from jax import lax
from jax.experimental import pallas as pl
from jax.experimental.pallas import tpu as pltpu

_B, _C, _L = 256, 20, 4096
_TL = 4096
_BB = 4

_ROTS = ((13, 15, 26, 6), (17, 29, 16, 24))


def _np_threefry2x32(k0, k1, x0, x1):
    mask = np.uint64(0xFFFFFFFF)
    x0 = np.asarray(x0, np.uint64)
    x1 = np.asarray(x1, np.uint64)

    def rotl(x, d):
        return ((x << np.uint64(d)) | (x >> np.uint64(32 - d))) & mask

    ks0, ks1 = np.uint64(k0), np.uint64(k1)
    ks2 = ks0 ^ ks1 ^ np.uint64(0x1BD11BDA)
    x0 = (x0 + ks0) & mask
    x1 = (x1 + ks1) & mask
    inj = ((ks1, ks2, 1), (ks2, ks0, 2), (ks0, ks1, 3), (ks1, ks2, 4), (ks2, ks0, 5))
    for i, (a, b, inc) in enumerate(inj):
        for r in _ROTS[i % 2]:
            x0 = (x0 + x1) & mask
            x1 = rotl(x1, r)
            x1 = x1 ^ x0
        x0 = (x0 + a) & mask
        x1 = (x1 + b + np.uint64(inc)) & mask
    return x0.astype(np.uint32), x1.astype(np.uint32)


_o0, _o1 = _np_threefry2x32(0, 42, [0, 0], [0, 1])
_KS0, _KS1 = int(_o0[0]), int(_o1[0])


def _sample_kernel(x_ref, o_ref, rk_ref, cm_ref):
    b = pl.program_id(0)
    shp = (_BB, _C, _TL)

    ks0 = np.uint32(_KS0)
    ks1 = np.uint32(_KS1)
    ks2 = np.uint32(_KS0 ^ _KS1 ^ 0x1BD11BDA)

    @pl.when(b == 0)
    def _init():
        b_iota = lax.broadcasted_iota(jnp.uint32, shp, 0)
        l_iota = lax.broadcasted_iota(jnp.uint32, shp, 2)
        c_iota = lax.broadcasted_iota(jnp.uint32, shp, 1)
        rk_ref[...] = (b_iota * np.uint32(_L * _C)
                       + l_iota * np.uint32(_C) + c_iota + ks1)
        cm_ref[...] = lax.broadcasted_iota(jnp.int32, shp, 1)

    def rotl(x, d):
        return lax.shift_left(x, np.uint32(d)) | lax.shift_right_logical(
            x, np.uint32(32 - d))

    base = (b * (_BB * _L * _C)).astype(jnp.uint32)
    x1 = rk_ref[...] + base
    x0 = jnp.zeros_like(x1) + ks0
    inj = ((ks1, ks2, 1), (ks2, ks0, 2), (ks0, ks1, 3), (ks1, ks2, 4), (ks2, ks0, 5))
    for i, (a, b_, inc) in enumerate(inj):
        for r in _ROTS[i % 2]:
            x0 = x0 + x1
            x1 = rotl(x1, r)
            x1 = x1 ^ x0
        x0 = x0 + a
        x1 = x1 + b_ + np.uint32(inc)
    bits = x0 ^ x1

    fb = lax.shift_right_logical(bits, np.uint32(9)) | np.uint32(0x3F800000)
    tiny = np.float32(np.finfo(np.float32).tiny)
    u = jnp.maximum(lax.bitcast_convert_type(fb, jnp.float32) - np.float32(1.0),
                    tiny)
    g = -jnp.log(-jnp.log(u))

    vals = x_ref[...] + g
    maxv = jnp.max(vals, axis=1, keepdims=True)
    cm = cm_ref[...]
    idx = jnp.min(jnp.where(vals == maxv, cm, _C), axis=1, keepdims=True)
    o_ref[...] = (cm == idx).astype(jnp.float32)


def kernel(logits):
    return pl.pallas_call(
        _sample_kernel,
        grid=(_B // _BB,),
        in_specs=[pl.BlockSpec((_BB, _C, _TL), lambda b: (b, 0, 0))],
        out_specs=pl.BlockSpec((_BB, _C, _TL), lambda b: (b, 0, 0)),
        out_shape=jax.ShapeDtypeStruct((_B, _C, _L), jnp.float32),
        scratch_shapes=[
            pltpu.VMEM((_BB, _C, _TL), jnp.uint32),
            pltpu.VMEM((_BB, _C, _TL), jnp.int32),
        ],
        compiler_params=pltpu.CompilerParams(
            dimension_semantics=("arbitrary",)),
    )(logits)
